# baseline (speedup 1.0000x reference)
import functools

import jax
import jax.numpy as jnp
from jax import lax
from jax.experimental import pallas as pl
from jax.experimental.pallas import tpu as pltpu
from jax.experimental.pallas import tpu_sc as plsc

B = 4096
N = 20
DIM = 64
NC = 2
NS = 16
NW = NC * NS
B_PER_W = B // NW
CB = 32
NCHUNK = B_PER_W // CB


def _dbow_body(doc_ids_hbm, tn_hbm, d_hbm, ot_hbm, out_hbm,
               idxd_v, idxt_v, dvec_v, orows_v, outv, sem):
    wid = lax.axis_index("s") * NC + lax.axis_index("c")
    base_b = wid * B_PER_W
    iota16 = lax.iota(jnp.int32, 16)

    for chunk in range(NCHUNK):
        b0 = base_b + chunk * CB
        pltpu.sync_copy(doc_ids_hbm.at[pl.ds(b0, CB)], idxd_v)
        pltpu.sync_copy(tn_hbm.at[pl.ds(b0 * N, CB * N)], idxt_v)
        pltpu.async_copy(d_hbm.at[idxd_v], dvec_v, sem).wait()
        pltpu.async_copy(ot_hbm.at[idxt_v], orows_v, sem).wait()

        for g in range(CB // 16):
            blane = g * 16 + iota16
            rows_o = [blane * N + n for n in range(N)]

            def dbody(d, accs, blane=blane, rows_o=rows_o):
                dsp = jnp.full((16,), d, jnp.int32)
                dv = plsc.load_gather(dvec_v, [blane, dsp])
                return tuple(
                    acc + dv * plsc.load_gather(orows_v, [rows_o[n], dsp])
                    for n, acc in enumerate(accs))

            accs = lax.fori_loop(
                0, DIM, dbody,
                tuple(jnp.zeros((16,), jnp.float32) for _ in range(N)))
            for n in range(N):
                plsc.store_scatter(
                    outv, [blane, jnp.full((16,), n, jnp.int32)], accs[n])

        pltpu.sync_copy(outv, out_hbm.at[pl.ds(b0, CB)])


@jax.jit
def kernel(doc_ids, target_noise_ids, D, O):
    ot = O.T
    tn_flat = target_noise_ids.reshape(-1)
    mesh = plsc.VectorSubcoreMesh(core_axis_name="c", subcore_axis_name="s")
    run = pl.kernel(
        _dbow_body,
        out_type=jax.ShapeDtypeStruct((B, N), jnp.float32),
        mesh=mesh,
        scratch_types=[
            pltpu.VMEM((CB,), jnp.int32),
            pltpu.VMEM((CB * N,), jnp.int32),
            pltpu.VMEM((CB, DIM), jnp.float32),
            pltpu.VMEM((CB * N, DIM), jnp.float32),
            pltpu.VMEM((CB, N), jnp.float32),
            pltpu.SemaphoreType.DMA,
        ],
        compiler_params=pltpu.CompilerParams(
            needs_layout_passes=False, use_tc_tiling_on_sc=False),
    )
    return run(doc_ids, tn_flat, D, ot)


# speedup vs baseline: 1.0267x; 1.0267x over previous
import jax
import jax.numpy as jnp
from jax import lax
from jax.experimental import pallas as pl
from jax.experimental.pallas import tpu as pltpu
from jax.experimental.pallas import tpu_sc as plsc

B = 4096
N = 20
DIM = 64
W = 100000
NC = 2
NS = 16
NW = NC * NS
B_PER_W = B // NW
CB = 32
NCHUNK = B_PER_W // CB
TBLK = 6400


def _transpose_body(o_ref, ot_ref):
    ot_ref[...] = o_ref[...].T


def _transpose(o):
    return pl.pallas_call(
        _transpose_body,
        grid=(pl.cdiv(W, TBLK),),
        in_specs=[pl.BlockSpec((DIM, TBLK), lambda i: (0, i))],
        out_specs=pl.BlockSpec((TBLK, DIM), lambda i: (i, 0)),
        out_shape=jax.ShapeDtypeStruct((W, DIM), jnp.float32),
    )(o)


def _dbow_body(doc_ids_hbm, tn_hbm, d_hbm, ot_hbm, out_hbm,
               idxd_v, idxt_v, dvec_v, orows_v, outv,
               sem_d, sem_o0, sem_o1):
    wid = lax.axis_index("s") * NC + lax.axis_index("c")
    base_b = wid * B_PER_W
    iota16 = lax.iota(jnp.int32, 16)
    sems = [sem_o0, sem_o1]

    pltpu.sync_copy(doc_ids_hbm.at[pl.ds(base_b, B_PER_W)], idxd_v)
    pltpu.sync_copy(tn_hbm.at[pl.ds(base_b * N, B_PER_W * N)], idxt_v)
    cp_d = pltpu.async_copy(d_hbm.at[idxd_v], dvec_v, sem_d)
    cps = [None, None]
    cps[0] = pltpu.async_copy(
        ot_hbm.at[idxt_v.at[pl.ds(0, CB * N)]], orows_v.at[0], sems[0])
    cp_d.wait()

    for c in range(NCHUNK):
        buf = c % 2
        cps[buf].wait()
        if c + 1 < NCHUNK:
            nbuf = (c + 1) % 2
            cps[nbuf] = pltpu.async_copy(
                ot_hbm.at[idxt_v.at[pl.ds((c + 1) * CB * N, CB * N)]],
                orows_v.at[nbuf], sems[nbuf])

        for g in range(CB // 16):
            blane = c * CB + g * 16 + iota16
            olane = g * 16 + iota16
            rows_o = [olane * N + n for n in range(N)]

            def dbody(d, accs, blane=blane, rows_o=rows_o, buf=buf):
                dsp = jnp.full((16,), d, jnp.int32)
                dv = plsc.load_gather(dvec_v, [blane, dsp])
                return tuple(
                    acc + dv * plsc.load_gather(orows_v.at[buf],
                                                [rows_o[n], dsp])
                    for n, acc in enumerate(accs))

            accs = lax.fori_loop(
                0, DIM, dbody,
                tuple(jnp.zeros((16,), jnp.float32) for _ in range(N)))
            for n in range(N):
                plsc.store_scatter(
                    outv, [blane, jnp.full((16,), n, jnp.int32)], accs[n])

    pltpu.sync_copy(outv, out_hbm.at[pl.ds(base_b, B_PER_W)])


@jax.jit
def kernel(doc_ids, target_noise_ids, D, O):
    ot = _transpose(O)
    tn_flat = target_noise_ids.reshape(-1)
    mesh = plsc.VectorSubcoreMesh(core_axis_name="c", subcore_axis_name="s")
    run = pl.kernel(
        _dbow_body,
        out_type=jax.ShapeDtypeStruct((B, N), jnp.float32),
        mesh=mesh,
        scratch_types=[
            pltpu.VMEM((B_PER_W,), jnp.int32),
            pltpu.VMEM((B_PER_W * N,), jnp.int32),
            pltpu.VMEM((B_PER_W, DIM), jnp.float32),
            pltpu.VMEM((2, CB * N, DIM), jnp.float32),
            pltpu.VMEM((B_PER_W, N), jnp.float32),
            pltpu.SemaphoreType.DMA,
            pltpu.SemaphoreType.DMA,
            pltpu.SemaphoreType.DMA,
        ],
        compiler_params=pltpu.CompilerParams(
            needs_layout_passes=False, use_tc_tiling_on_sc=False),
    )
    return run(doc_ids, tn_flat, D, ot)


# speedup vs baseline: 1.0502x; 1.0229x over previous
import jax
import jax.numpy as jnp
from jax import lax
from jax.experimental import pallas as pl
from jax.experimental.pallas import tpu as pltpu
from jax.experimental.pallas import tpu_sc as plsc

B = 4096
N = 20
DIM = 64
W = 100000
NC = 2
NS = 16
NW = NC * NS
B_PER_W = B // NW
CB = 32
NCHUNK = B_PER_W // CB
CC = 320
NFULL = W // CC
TAIL = W - NFULL * CC
TAIL_W = (NFULL % NW)


def _transpose_body(o_hbm, ot_hbm, in_v, out_v):
    wid = lax.axis_index("s") * NC + lax.axis_index("c")
    iota16 = lax.iota(jnp.int32, 16)

    def do_chunk(c0, cc):
        pltpu.sync_copy(o_hbm.at[:, pl.ds(c0, cc)],
                        in_v.at[:, pl.ds(0, cc)])

        def dbody(d, _):
            dsp = jnp.full((16,), d, jnp.int32)
            for t0 in range(0, cc, 16):
                v = in_v[d, pl.ds(t0, 16)]
                plsc.store_scatter(out_v, [t0 + iota16, dsp], v)
            return 0

        lax.fori_loop(0, DIM, dbody, 0)
        pltpu.sync_copy(out_v.at[pl.ds(0, cc)], ot_hbm.at[pl.ds(c0, cc)])

    for j in range((NFULL + NW - 1) // NW):
        cid = wid + NW * j
        @pl.when(cid < NFULL)
        def _():
            do_chunk(cid * CC, CC)

    @pl.when(wid == TAIL_W)
    def _():
        do_chunk(NFULL * CC, TAIL)


def _transpose(o):
    mesh = plsc.VectorSubcoreMesh(core_axis_name="c", subcore_axis_name="s")
    return pl.kernel(
        _transpose_body,
        out_type=jax.ShapeDtypeStruct((W, DIM), jnp.float32),
        mesh=mesh,
        scratch_types=[
            pltpu.VMEM((DIM, CC), jnp.float32),
            pltpu.VMEM((CC, DIM), jnp.float32),
        ],
        compiler_params=pltpu.CompilerParams(
            needs_layout_passes=False, use_tc_tiling_on_sc=False),
    )(o)


def _dbow_body(doc_ids_hbm, tn_hbm, d_hbm, ot_hbm, out_hbm,
               idxd_v, idxt_v, dvec_v, orows_v, outv,
               sem_d, sem_o0, sem_o1):
    wid = lax.axis_index("s") * NC + lax.axis_index("c")
    base_b = wid * B_PER_W
    iota16 = lax.iota(jnp.int32, 16)
    sems = [sem_o0, sem_o1]

    pltpu.sync_copy(doc_ids_hbm.at[pl.ds(base_b, B_PER_W)], idxd_v)
    pltpu.sync_copy(tn_hbm.at[pl.ds(base_b * N, B_PER_W * N)], idxt_v)
    cp_d = pltpu.async_copy(d_hbm.at[idxd_v], dvec_v, sem_d)
    cps = [None, None]
    cps[0] = pltpu.async_copy(
        ot_hbm.at[idxt_v.at[pl.ds(0, CB * N)]], orows_v.at[0], sems[0])
    cp_d.wait()

    for c in range(NCHUNK):
        buf = c % 2
        cps[buf].wait()
        if c + 1 < NCHUNK:
            nbuf = (c + 1) % 2
            cps[nbuf] = pltpu.async_copy(
                ot_hbm.at[idxt_v.at[pl.ds((c + 1) * CB * N, CB * N)]],
                orows_v.at[nbuf], sems[nbuf])

        for g in range(CB // 16):
            blane = c * CB + g * 16 + iota16
            olane = g * 16 + iota16
            rows_o = [olane * N + n for n in range(N)]

            def dbody(d, accs, blane=blane, rows_o=rows_o, buf=buf):
                dsp = jnp.full((16,), d, jnp.int32)
                dv = plsc.load_gather(dvec_v, [blane, dsp])
                return tuple(
                    acc + dv * plsc.load_gather(orows_v.at[buf],
                                                [rows_o[n], dsp])
                    for n, acc in enumerate(accs))

            accs = lax.fori_loop(
                0, DIM, dbody,
                tuple(jnp.zeros((16,), jnp.float32) for _ in range(N)))
            for n in range(N):
                plsc.store_scatter(
                    outv, [blane, jnp.full((16,), n, jnp.int32)], accs[n])

    pltpu.sync_copy(outv, out_hbm.at[pl.ds(base_b, B_PER_W)])


@jax.jit
def kernel(doc_ids, target_noise_ids, D, O):
    ot = _transpose(O)
    tn_flat = target_noise_ids.reshape(-1)
    mesh = plsc.VectorSubcoreMesh(core_axis_name="c", subcore_axis_name="s")
    run = pl.kernel(
        _dbow_body,
        out_type=jax.ShapeDtypeStruct((B, N), jnp.float32),
        mesh=mesh,
        scratch_types=[
            pltpu.VMEM((B_PER_W,), jnp.int32),
            pltpu.VMEM((B_PER_W * N,), jnp.int32),
            pltpu.VMEM((B_PER_W, DIM), jnp.float32),
            pltpu.VMEM((2, CB * N, DIM), jnp.float32),
            pltpu.VMEM((B_PER_W, N), jnp.float32),
            pltpu.SemaphoreType.DMA,
            pltpu.SemaphoreType.DMA,
            pltpu.SemaphoreType.DMA,
        ],
        compiler_params=pltpu.CompilerParams(
            needs_layout_passes=False, use_tc_tiling_on_sc=False),
    )
    return run(doc_ids, tn_flat, D, ot)
